# baseline (speedup 1.0000x reference)
import functools

import jax
import jax.numpy as jnp
from jax import lax
from jax.experimental import pallas as pl
from jax.experimental.pallas import tpu as pltpu
from jax.experimental.pallas import tpu_sc as plsc

_VOCAB = 1000000
_D = 64
_BATCH = 16384
_HIST = 50
_TOTAL = _BATCH * _HIST
_NC = 2
_NS = 16
_NW = _NC * _NS
_PER_W = _TOTAL // _NW
_CH = 512
_CHH = _CH // 2
_NCHUNK = _PER_W // _CH

_mesh = plsc.VectorSubcoreMesh(core_axis_name="c", subcore_axis_name="s")


@functools.partial(
    pl.kernel,
    mesh=_mesh,
    out_type=jax.ShapeDtypeStruct((_TOTAL * _D // 128, 128), jnp.float32),
    scratch_types=[
        pltpu.VMEM((_PER_W,), jnp.int32),
        pltpu.VMEM((_CHH, _D), jnp.float32),
        pltpu.VMEM((_CHH, _D), jnp.float32),
        pltpu.VMEM((_CHH, _D), jnp.float32),
        pltpu.VMEM((_CHH, _D), jnp.float32),
        pltpu.SemaphoreType.DMA,
        pltpu.SemaphoreType.DMA,
    ],
    compiler_params=pltpu.CompilerParams(use_tc_tiling_on_sc=False),
)
def _gather_kernel(
    idx2_hbm, table2m_hbm, out128_hbm, idx_v, ev0, od0, ev1, od1, g0, g1
):
    wid = lax.axis_index("s") * _NC + lax.axis_index("c")
    base = wid * _PER_W
    pltpu.sync_copy(idx2_hbm.at[pl.ds(base, _PER_W)], idx_v)

    def gather_pair(c, bufe, bufo, sem, start):
        maker = pltpu.async_copy if start else pltpu.make_async_copy
        de = maker(
            table2m_hbm.at[idx_v.at[pl.ds(c * _CH, _CHH)]], bufe, sem
        )
        do = maker(
            table2m_hbm.at[idx_v.at[pl.ds(c * _CH + _CHH, _CHH)]], bufo, sem
        )
        if not start:
            de.wait()
            do.wait()

    gather_pair(0, ev0, od0, g0, True)
    gather_pair(1, ev1, od1, g1, True)

    def handle(c, bufe, bufo, sem):
        gather_pair(c, bufe, bufo, sem, False)
        j0 = (base + c * _CH) // 2
        pltpu.sync_copy(bufe, out128_hbm.at[pl.ds(j0, _CHH), pl.ds(0, _D)])
        pltpu.sync_copy(bufo, out128_hbm.at[pl.ds(j0, _CHH), pl.ds(_D, _D)])

        @pl.when(c + 2 < _NCHUNK)
        def _():
            gather_pair(c + 2, bufe, bufo, sem, True)

    def body(p, carry):
        handle(2 * p, ev0, od0, g0)
        handle(2 * p + 1, ev1, od1, g1)
        return carry

    lax.fori_loop(0, _NCHUNK // 2, body, 0)


def kernel(input, table):
    idx2 = (
        (input.reshape(_TOTAL) * 2)
        .reshape(_TOTAL // _CH, _CHH, 2)
        .transpose(0, 2, 1)
        .reshape(_TOTAL)
    )
    table2m = jnp.pad(table, ((0, 0), (0, 128 - _D))).reshape(2 * _VOCAB, _D)
    out = _gather_kernel(idx2, table2m)
    return out.reshape(_BATCH, _HIST, _D)


# speedup vs baseline: 1.1211x; 1.1211x over previous
import functools

import jax
import jax.numpy as jnp
from jax import lax
from jax.experimental import pallas as pl
from jax.experimental.pallas import tpu as pltpu
from jax.experimental.pallas import tpu_sc as plsc

_VOCAB = 1000000
_D = 64
_BATCH = 16384
_HIST = 50
_TOTAL = _BATCH * _HIST
_NC = 2
_NS = 16
_NW = _NC * _NS
_PER_W = _TOTAL // _NW
_CH = 512
_NCHUNK = _PER_W // _CH

_mesh = plsc.VectorSubcoreMesh(core_axis_name="c", subcore_axis_name="s")


@functools.partial(
    pl.kernel,
    mesh=_mesh,
    out_type=jax.ShapeDtypeStruct((_TOTAL, _D), jnp.float32),
    scratch_types=[
        pltpu.VMEM((_PER_W,), jnp.int32),
        pltpu.VMEM((_CH, _D), jnp.float32),
        pltpu.VMEM((_CH, _D), jnp.float32),
        pltpu.SemaphoreType.DMA,
        pltpu.SemaphoreType.DMA,
    ],
    compiler_params=pltpu.CompilerParams(use_tc_tiling_on_sc=False),
)
def _gather_kernel(idx2_hbm, table2m_hbm, out_hbm, idx_v, buf0, buf1, g0, g1):
    wid = lax.axis_index("s") * _NC + lax.axis_index("c")
    base = wid * _PER_W
    pltpu.sync_copy(idx2_hbm.at[pl.ds(base, _PER_W)], idx_v)

    def gather_start(c, buf, sem):
        pltpu.async_copy(table2m_hbm.at[idx_v.at[pl.ds(c * _CH, _CH)]], buf, sem)

    def gather_wait(c, buf, sem):
        pltpu.make_async_copy(
            table2m_hbm.at[idx_v.at[pl.ds(c * _CH, _CH)]], buf, sem
        ).wait()

    gather_start(0, buf0, g0)
    gather_start(1, buf1, g1)

    def handle(c, buf, sem):
        gather_wait(c, buf, sem)
        pltpu.sync_copy(buf, out_hbm.at[pl.ds(base + c * _CH, _CH)])

        @pl.when(c + 2 < _NCHUNK)
        def _():
            gather_start(c + 2, buf, sem)

    def body(p, carry):
        handle(2 * p, buf0, g0)
        handle(2 * p + 1, buf1, g1)
        return carry

    lax.fori_loop(0, _NCHUNK // 2, body, 0)


def kernel(input, table):
    idx2 = input.reshape(_TOTAL) * 2
    table2m = jnp.pad(table, ((0, 0), (0, 128 - _D))).reshape(2 * _VOCAB, _D)
    out = _gather_kernel(idx2, table2m)
    return out.reshape(_TOTAL // 2, 2 * _D).reshape(_BATCH, _HIST, _D)
